# baseline (speedup 1.0000x reference)
import functools

import jax
import jax.numpy as jnp
from jax import lax
from jax.experimental import pallas as pl
from jax.experimental.pallas import tpu as pltpu
from jax.experimental.pallas import tpu_sc as plsc

N = 16777216
BINS = 1024
LANES = 16
NC = 2
NS = 16
NW = NC * NS
PER_W = N // NW
CHUNK = 16384
NCHUNK = PER_W // CHUNK
VPC = CHUNK // LANES
STRIDE = BINS + 1


def _hist_body(x_hbm, lo_hbm, scale_hbm, out_hbm, xb0, xb1, xb2, xb3,
               hist, sums, lo_v, scale_v, sem0, sem1, sem2, sem3):
    wid = lax.axis_index("s") * NC + lax.axis_index("c")
    base = wid * PER_W

    def copy(g, buf, sem):
        return pltpu.make_async_copy(
            x_hbm.at[pl.ds(base + g * CHUNK, CHUNK)], buf, sem)

    bufs = ((xb0, sem0), (xb1, sem1), (xb2, sem2), (xb3, sem3))
    for g0, (buf0, sem0_) in enumerate(bufs):
        copy(g0, buf0, sem0_).start()

    pltpu.sync_copy(lo_hbm, lo_v)
    pltpu.sync_copy(scale_hbm, scale_v)
    lo = lo_v[...]
    scale = scale_v[...]
    nls = jnp.zeros((LANES,), jnp.float32) - lo * scale
    lane_off = lax.iota(jnp.int32, LANES) * STRIDE
    zeros16 = jnp.zeros((LANES,), jnp.float32)
    ones16 = jnp.full((LANES,), 1.0, jnp.float32)

    @plsc.parallel_loop(0, (LANES * STRIDE + LANES - 1) // LANES, unroll=8)
    def _(i):
        hist[pl.ds(i * LANES, LANES)] = zeros16

    def chunk_body(g4, _):
        for b, (buf, sem) in enumerate(bufs):
            g = 4 * g4 + b
            copy(g, buf, sem).wait()

            @plsc.parallel_loop(0, VPC, unroll=8)
            def _(i):
                v = buf[pl.ds(i * LANES, LANES)]
                idx = (v * scale + nls).astype(jnp.int32)
                plsc.addupdate_scatter(hist, [idx + lane_off], ones16)

            @pl.when(g + 4 < NCHUNK)
            def _():
                copy(g + 4, buf, sem).start()
        return 0

    lax.fori_loop(0, NCHUNK // 4, chunk_body, 0)

    def red_body(cb, _):
        acc = hist[pl.ds(cb * LANES, LANES)]
        for r in range(1, LANES):
            acc = acc + hist[pl.ds(r * STRIDE + cb * LANES, LANES)]
        sums[pl.ds(cb * LANES, LANES)] = acc
        return 0

    lax.fori_loop(0, BINS // LANES, red_body, 0)

    spare = plsc.load_gather(hist, [lane_off + (STRIDE - 1)])
    spare_tot = jnp.sum(spare)
    is_last = lax.iota(jnp.int32, LANES) == (LANES - 1)
    corr = jnp.where(is_last, jnp.broadcast_to(spare_tot, (LANES,)), zeros16)
    tail = sums[pl.ds(BINS - LANES, LANES)]
    sums[pl.ds(BINS - LANES, LANES)] = tail + corr
    pltpu.sync_copy(sums, out_hbm.at[wid])


@functools.partial(jax.jit, static_argnums=(1,))
def _histc(x, bins, lo, scale):
    mesh = plsc.VectorSubcoreMesh(core_axis_name="c", subcore_axis_name="s",
                                  num_cores=NC, num_subcores=NS)
    call = pl.kernel(
        _hist_body,
        out_type=jax.ShapeDtypeStruct((NW, BINS), jnp.float32),
        mesh=mesh,
        scratch_types=[
            pltpu.VMEM((CHUNK,), jnp.float32),
            pltpu.VMEM((CHUNK,), jnp.float32),
            pltpu.VMEM((CHUNK,), jnp.float32),
            pltpu.VMEM((CHUNK,), jnp.float32),
            pltpu.VMEM((LANES * STRIDE,), jnp.float32),
            pltpu.VMEM((BINS,), jnp.float32),
            pltpu.VMEM((LANES,), jnp.float32),
            pltpu.VMEM((LANES,), jnp.float32),
            pltpu.SemaphoreType.DMA,
            pltpu.SemaphoreType.DMA,
            pltpu.SemaphoreType.DMA,
            pltpu.SemaphoreType.DMA,
        ],
        compiler_params=pltpu.CompilerParams(needs_layout_passes=False),
    )
    lo16 = jnp.full((LANES,), lo, jnp.float32)
    scale16 = jnp.full((LANES,), scale, jnp.float32)
    partial = call(x, lo16, scale16)
    return jnp.sum(partial, axis=0)


def kernel(x, bins, min, max, out):
    lo = jnp.float32(min)
    hi = jnp.float32(max)
    width = hi - lo
    scale = jnp.float32(bins) / width
    return _histc(x, out.shape[0], lo, scale)


# speedup vs baseline: 1.0288x; 1.0288x over previous
import functools

import jax
import jax.numpy as jnp
from jax import lax
from jax.experimental import pallas as pl
from jax.experimental.pallas import tpu as pltpu
from jax.experimental.pallas import tpu_sc as plsc

N = 16777216
BINS = 1024
LANES = 16
NC = 2
NS = 16
NW = NC * NS
PER_W = N // NW
CHUNK = 16384
NCHUNK = PER_W // CHUNK
VPC = CHUNK // LANES
STRIDE = BINS + 1


def _hist_body(x_hbm, lo_hbm, scale_hbm, out_hbm, xb0, xb1, hist, sums,
               lo_v, scale_v, sem0, sem1, semp):
    wid = lax.axis_index("s") * NC + lax.axis_index("c")
    base = wid * PER_W

    def copy(g, buf, sem):
        return pltpu.make_async_copy(
            x_hbm.at[pl.ds(base + g * CHUNK, CHUNK)], buf, sem)

    bufs = ((xb0, sem0), (xb1, sem1))
    copy(0, xb0, sem0).start()
    copy(1, xb1, sem1).start()

    cp_lo = pltpu.make_async_copy(lo_hbm, lo_v, semp)
    cp_sc = pltpu.make_async_copy(scale_hbm, scale_v, semp)
    cp_lo.start()
    cp_sc.start()
    lane_off = lax.iota(jnp.int32, LANES) * STRIDE
    zeros16 = jnp.zeros((LANES,), jnp.float32)
    ones16 = jnp.full((LANES,), 1.0, jnp.float32)

    @plsc.parallel_loop(0, (LANES * STRIDE + LANES - 1) // LANES, unroll=8)
    def _(i):
        hist[pl.ds(i * LANES, LANES)] = zeros16

    cp_lo.wait()
    cp_sc.wait()
    lo = lo_v[...]
    scale = scale_v[...]
    nls = jnp.zeros((LANES,), jnp.float32) - lo * scale

    def chunk_body(g2, _):
        for b, (buf, sem) in enumerate(bufs):
            g = 2 * g2 + b
            copy(g, buf, sem).wait()

            @plsc.parallel_loop(0, VPC, unroll=8)
            def _(i):
                v = buf[pl.ds(i * LANES, LANES)]
                idx = (v * scale + nls).astype(jnp.int32)
                plsc.addupdate_scatter(hist, [idx + lane_off], ones16)

            @pl.when(g + 2 < NCHUNK)
            def _():
                copy(g + 2, buf, sem).start()
        return 0

    lax.fori_loop(0, NCHUNK // 2, chunk_body, 0)

    def red_body(cb, _):
        acc = hist[pl.ds(cb * LANES, LANES)]
        for r in range(1, LANES):
            acc = acc + hist[pl.ds(r * STRIDE + cb * LANES, LANES)]
        sums[pl.ds(cb * LANES, LANES)] = acc
        return 0

    lax.fori_loop(0, BINS // LANES, red_body, 0)

    spare = plsc.load_gather(hist, [lane_off + (STRIDE - 1)])
    spare_tot = jnp.sum(spare)
    is_last = lax.iota(jnp.int32, LANES) == (LANES - 1)
    corr = jnp.where(is_last, jnp.broadcast_to(spare_tot, (LANES,)), zeros16)
    tail = sums[pl.ds(BINS - LANES, LANES)]
    sums[pl.ds(BINS - LANES, LANES)] = tail + corr
    pltpu.sync_copy(sums, out_hbm.at[wid])


@functools.partial(jax.jit, static_argnums=(1,))
def _histc(x, bins, lo, scale):
    mesh = plsc.VectorSubcoreMesh(core_axis_name="c", subcore_axis_name="s",
                                  num_cores=NC, num_subcores=NS)
    call = pl.kernel(
        _hist_body,
        out_type=jax.ShapeDtypeStruct((NW, BINS), jnp.float32),
        mesh=mesh,
        scratch_types=[
            pltpu.VMEM((CHUNK,), jnp.float32),
            pltpu.VMEM((CHUNK,), jnp.float32),
            pltpu.VMEM((LANES * STRIDE,), jnp.float32),
            pltpu.VMEM((BINS,), jnp.float32),
            pltpu.VMEM((LANES,), jnp.float32),
            pltpu.VMEM((LANES,), jnp.float32),
            pltpu.SemaphoreType.DMA,
            pltpu.SemaphoreType.DMA,
            pltpu.SemaphoreType.DMA,
        ],
        compiler_params=pltpu.CompilerParams(needs_layout_passes=False),
    )
    lo16 = jnp.full((LANES,), lo, jnp.float32)
    scale16 = jnp.full((LANES,), scale, jnp.float32)
    partial = call(x, lo16, scale16)
    return jnp.sum(partial, axis=0)


def kernel(x, bins, min, max, out):
    lo = jnp.float32(min)
    hi = jnp.float32(max)
    width = hi - lo
    scale = jnp.float32(bins) / width
    return _histc(x, out.shape[0], lo, scale)


# speedup vs baseline: 1.0466x; 1.0172x over previous
import functools

import jax
import jax.numpy as jnp
from jax import lax
from jax.experimental import pallas as pl
from jax.experimental.pallas import tpu as pltpu
from jax.experimental.pallas import tpu_sc as plsc

N = 16777216
BINS = 1024
LANES = 16
NC = 2
NS = 16
NW = NC * NS
PER_W = N // NW
CHUNK = 16384
NCHUNK = PER_W // CHUNK
VPC = CHUNK // LANES
STRIDE = BINS + 1


def _hist_body(x_hbm, lo_hbm, scale_hbm, out_hbm, xb0, xb1, hist, sums,
               lo_v, scale_v, sem0, sem1, semp):
    wid = lax.axis_index("s") * NC + lax.axis_index("c")
    base = wid * PER_W

    def copy(g, buf, sem):
        return pltpu.make_async_copy(
            x_hbm.at[pl.ds(base + g * CHUNK, CHUNK)], buf, sem)

    bufs = ((xb0, sem0), (xb1, sem1))
    copy(0, xb0, sem0).start()
    copy(1, xb1, sem1).start()

    cp_lo = pltpu.make_async_copy(lo_hbm, lo_v, semp)
    cp_sc = pltpu.make_async_copy(scale_hbm, scale_v, semp)
    cp_lo.start()
    cp_sc.start()
    lane_off = lax.iota(jnp.int32, LANES) * STRIDE
    zeros16 = jnp.zeros((LANES,), jnp.float32)
    ones16 = jnp.full((LANES,), 1.0, jnp.float32)

    @plsc.parallel_loop(0, (LANES * STRIDE + LANES - 1) // LANES, unroll=8)
    def _(i):
        hist[pl.ds(i * LANES, LANES)] = zeros16

    cp_lo.wait()
    cp_sc.wait()
    lo = lo_v[...]
    scale = scale_v[...]
    nls = jnp.zeros((LANES,), jnp.float32) - lo * scale

    def chunk_body(g2, _):
        for b, (buf, sem) in enumerate(bufs):
            g = 2 * g2 + b
            copy(g, buf, sem).wait()

            @plsc.parallel_loop(0, VPC, unroll=4)
            def _(i):
                v = buf[pl.ds(i * LANES, LANES)]
                idx = (v * scale + nls).astype(jnp.int32)
                plsc.addupdate_scatter(hist, [idx + lane_off], ones16)

            @pl.when(g + 2 < NCHUNK)
            def _():
                copy(g + 2, buf, sem).start()
        return 0

    lax.fori_loop(0, NCHUNK // 2, chunk_body, 0)

    def red_body(cb, _):
        acc = hist[pl.ds(cb * LANES, LANES)]
        for r in range(1, LANES):
            acc = acc + hist[pl.ds(r * STRIDE + cb * LANES, LANES)]
        sums[pl.ds(cb * LANES, LANES)] = acc
        return 0

    lax.fori_loop(0, BINS // LANES, red_body, 0)

    spare = plsc.load_gather(hist, [lane_off + (STRIDE - 1)])
    spare_tot = jnp.sum(spare)
    is_last = lax.iota(jnp.int32, LANES) == (LANES - 1)
    corr = jnp.where(is_last, jnp.broadcast_to(spare_tot, (LANES,)), zeros16)
    tail = sums[pl.ds(BINS - LANES, LANES)]
    sums[pl.ds(BINS - LANES, LANES)] = tail + corr
    pltpu.sync_copy(sums, out_hbm.at[wid])


@functools.partial(jax.jit, static_argnums=(1,))
def _histc(x, bins, lo, scale):
    mesh = plsc.VectorSubcoreMesh(core_axis_name="c", subcore_axis_name="s",
                                  num_cores=NC, num_subcores=NS)
    call = pl.kernel(
        _hist_body,
        out_type=jax.ShapeDtypeStruct((NW, BINS), jnp.float32),
        mesh=mesh,
        scratch_types=[
            pltpu.VMEM((CHUNK,), jnp.float32),
            pltpu.VMEM((CHUNK,), jnp.float32),
            pltpu.VMEM((LANES * STRIDE,), jnp.float32),
            pltpu.VMEM((BINS,), jnp.float32),
            pltpu.VMEM((LANES,), jnp.float32),
            pltpu.VMEM((LANES,), jnp.float32),
            pltpu.SemaphoreType.DMA,
            pltpu.SemaphoreType.DMA,
            pltpu.SemaphoreType.DMA,
        ],
        compiler_params=pltpu.CompilerParams(needs_layout_passes=False),
    )
    lo16 = jnp.full((LANES,), lo, jnp.float32)
    scale16 = jnp.full((LANES,), scale, jnp.float32)
    partial = call(x, lo16, scale16)
    return jnp.sum(partial, axis=0)


def kernel(x, bins, min, max, out):
    lo = jnp.float32(min)
    hi = jnp.float32(max)
    width = hi - lo
    scale = jnp.float32(bins) / width
    return _histc(x, out.shape[0], lo, scale)
